# baseline (speedup 1.0000x reference)
import functools

import jax
import jax.numpy as jnp
from jax import lax
from jax.experimental import pallas as pl
from jax.experimental.pallas import tpu as pltpu
from jax.experimental.pallas import tpu_sc as plsc

NUM_WORDS = 1000000
NUM_SUBWORDS = 5000
D_EMBED = 64
MAX_SUBWORDS = 5
BATCH = 16384

NUM_CORES = 2
NUM_SUBCORES = 16
NUM_WORKERS = NUM_CORES * NUM_SUBCORES
B_PER_W = BATCH // NUM_WORKERS
CHUNK = 128
N_CHUNKS = B_PER_W // CHUNK
LANES = 16
N_SUB_CHUNKS = MAX_SUBWORDS * N_CHUNKS


def _body(word_hbm, wt_hbm, sub_hbm, w2sflat_hbm, out_hbm,
          widx, jbuf, craw, cidx, acc, tmp, sem):
    wid = lax.axis_index("s") * NUM_CORES + lax.axis_index("c")
    gbase = wid * B_PER_W

    for c in range(N_CHUNKS):
        pltpu.sync_copy(word_hbm.at[pl.ds(gbase + CHUNK * c, CHUNK)],
                        widx.at[c])

    for c in range(N_CHUNKS):
        pltpu.async_copy(wt_hbm.at[widx.at[c]],
                         acc.at[pl.ds(CHUNK * c, CHUNK)], sem).wait()

    for j in range(MAX_SUBWORDS):
        for c in range(N_CHUNKS):
            rj = N_CHUNKS * j + c
            for t in range(CHUNK // LANES):
                sl = pl.ds(LANES * t, LANES)
                jbuf[rj, sl] = widx[c, sl] * MAX_SUBWORDS + j

    for rj in range(N_SUB_CHUNKS):
        pltpu.async_copy(w2sflat_hbm.at[jbuf.at[rj]], craw.at[rj], sem).wait()

    for rj in range(N_SUB_CHUNKS):
        for t in range(CHUNK // LANES):
            sl = pl.ds(LANES * t, LANES)
            v = craw[rj, sl]
            cidx[rj, sl] = jnp.where(v == 0, jnp.int32(NUM_SUBWORDS), v)

    for j in range(MAX_SUBWORDS):
        for c in range(N_CHUNKS):
            pltpu.async_copy(sub_hbm.at[cidx.at[N_CHUNKS * j + c]],
                             tmp, sem).wait()

            def r_body(r, _, c=c):
                row = CHUNK * c + r
                for k in range(D_EMBED // LANES):
                    sl = pl.ds(LANES * k, LANES)
                    acc[row, sl] = acc[row, sl] + tmp[r, sl]
                return 0

            lax.fori_loop(0, CHUNK, r_body, 0)

    pltpu.sync_copy(acc, out_hbm.at[pl.ds(gbase, B_PER_W)])


_mesh = plsc.VectorSubcoreMesh(core_axis_name="c", subcore_axis_name="s")

_sc_embed = functools.partial(
    pl.kernel,
    out_type=jax.ShapeDtypeStruct((BATCH, D_EMBED), jnp.float32),
    mesh=_mesh,
    scratch_types=[
        pltpu.VMEM((N_CHUNKS, CHUNK), jnp.int32),
        pltpu.VMEM((N_SUB_CHUNKS, CHUNK), jnp.int32),
        pltpu.VMEM((N_SUB_CHUNKS, CHUNK), jnp.int32),
        pltpu.VMEM((N_SUB_CHUNKS, CHUNK), jnp.int32),
        pltpu.VMEM((B_PER_W, D_EMBED), jnp.float32),
        pltpu.VMEM((CHUNK, D_EMBED), jnp.float32),
        pltpu.SemaphoreType.DMA,
    ],
    compiler_params=pltpu.CompilerParams(use_tc_tiling_on_sc=False),
)(_body)


@jax.jit
def kernel(word, word_table, subword_table, word_to_subwords):
    word = word.astype(jnp.int32)
    w2s_flat = word_to_subwords.astype(jnp.int32).reshape(-1)
    sub_ext = jnp.concatenate(
        [subword_table, jnp.zeros((1, D_EMBED), subword_table.dtype)], axis=0)
    return _sc_embed(word, word_table, sub_ext, w2s_flat)


# speedup vs baseline: 1.0341x; 1.0341x over previous
import functools

import jax
import jax.numpy as jnp
from jax import lax
from jax.experimental import pallas as pl
from jax.experimental.pallas import tpu as pltpu
from jax.experimental.pallas import tpu_sc as plsc

NUM_WORDS = 1000000
NUM_SUBWORDS = 5000
D_EMBED = 64
MAX_SUBWORDS = 5
BATCH = 16384

NUM_CORES = 2
NUM_SUBCORES = 16
NUM_WORKERS = NUM_CORES * NUM_SUBCORES
B_PER_W = BATCH // NUM_WORKERS
CHUNK = 128
N_CHUNKS = B_PER_W // CHUNK
LANES = 16
N_SUB_CHUNKS = MAX_SUBWORDS * N_CHUNKS


def _body(word_hbm, wt_hbm, sub_hbm, w2sflat_hbm, out_hbm,
          widx, jbuf, cidx, acc, tmp_a, tmp_b,
          sem_w, sem_b, sem_s, sem_o):
    wid = lax.axis_index("s") * NUM_CORES + lax.axis_index("c")
    gbase = wid * B_PER_W

    for c in range(N_CHUNKS):
        pltpu.sync_copy(word_hbm.at[pl.ds(gbase + CHUNK * c, CHUNK)],
                        widx.at[c])

    wdescs = [
        pltpu.async_copy(wt_hbm.at[widx.at[c]],
                         acc.at[pl.ds(CHUNK * c, CHUNK)], sem_w)
        for c in range(N_CHUNKS)
    ]

    for j in range(MAX_SUBWORDS):
        for c in range(N_CHUNKS):
            rj = N_CHUNKS * j + c
            for t in range(CHUNK // LANES):
                sl = pl.ds(LANES * t, LANES)
                jbuf[rj, sl] = widx[c, sl] * MAX_SUBWORDS + j

    bdescs = [
        pltpu.async_copy(w2sflat_hbm.at[jbuf.at[rj]], cidx.at[rj], sem_b)
        for rj in range(N_SUB_CHUNKS)
    ]
    for d in bdescs:
        d.wait()

    for rj in range(N_SUB_CHUNKS):
        for t in range(CHUNK // LANES):
            sl = pl.ds(LANES * t, LANES)
            v = cidx[rj, sl]
            cidx[rj, sl] = jnp.where(v == 0, jnp.int32(NUM_SUBWORDS), v)

    bufs = (tmp_a, tmp_b)

    def fire(c, buf):
        return [
            pltpu.async_copy(sub_hbm.at[cidx.at[N_CHUNKS * j + c]],
                             buf.at[j], sem_s)
            for j in range(MAX_SUBWORDS)
        ]

    odescs = []
    pend = fire(0, bufs[0])
    for c in range(N_CHUNKS):
        for d in pend:
            d.wait()
        if c + 1 < N_CHUNKS:
            pend = fire(c + 1, bufs[(c + 1) % 2])
        wdescs[c].wait()
        buf = bufs[c % 2]

        def r_body(r, _, c=c, buf=buf):
            row = CHUNK * c + r
            for k in range(D_EMBED // LANES):
                sl = pl.ds(LANES * k, LANES)
                s01 = buf[0, r, sl] + buf[1, r, sl]
                s23 = buf[2, r, sl] + buf[3, r, sl]
                s = s01 + s23 + buf[4, r, sl]
                acc[row, sl] = acc[row, sl] + s
            return 0

        lax.fori_loop(0, CHUNK, r_body, 0)
        odescs.append(
            pltpu.async_copy(acc.at[pl.ds(CHUNK * c, CHUNK)],
                             out_hbm.at[pl.ds(gbase + CHUNK * c, CHUNK)],
                             sem_o))
    for d in odescs:
        d.wait()


_mesh = plsc.VectorSubcoreMesh(core_axis_name="c", subcore_axis_name="s")

_sc_embed = functools.partial(
    pl.kernel,
    out_type=jax.ShapeDtypeStruct((BATCH, D_EMBED), jnp.float32),
    mesh=_mesh,
    scratch_types=[
        pltpu.VMEM((N_CHUNKS, CHUNK), jnp.int32),
        pltpu.VMEM((N_SUB_CHUNKS, CHUNK), jnp.int32),
        pltpu.VMEM((N_SUB_CHUNKS, CHUNK), jnp.int32),
        pltpu.VMEM((B_PER_W, D_EMBED), jnp.float32),
        pltpu.VMEM((MAX_SUBWORDS, CHUNK, D_EMBED), jnp.float32),
        pltpu.VMEM((MAX_SUBWORDS, CHUNK, D_EMBED), jnp.float32),
        pltpu.SemaphoreType.DMA,
        pltpu.SemaphoreType.DMA,
        pltpu.SemaphoreType.DMA,
        pltpu.SemaphoreType.DMA,
    ],
    compiler_params=pltpu.CompilerParams(use_tc_tiling_on_sc=False),
)(_body)


@jax.jit
def kernel(word, word_table, subword_table, word_to_subwords):
    word = word.astype(jnp.int32)
    w2s_flat = word_to_subwords.astype(jnp.int32).reshape(-1)
    sub_ext = jnp.concatenate(
        [subword_table, jnp.zeros((1, D_EMBED), subword_table.dtype)], axis=0)
    return _sc_embed(word, word_table, sub_ext, w2s_flat)


# speedup vs baseline: 1.5640x; 1.5124x over previous
import functools

import jax
import jax.numpy as jnp
from jax import lax
from jax.experimental import pallas as pl
from jax.experimental.pallas import tpu as pltpu
from jax.experimental.pallas import tpu_sc as plsc

NUM_WORDS = 1000000
NUM_SUBWORDS = 5000
D_EMBED = 64
MAX_SUBWORDS = 5
BATCH = 16384

NUM_CORES = 2
NUM_SUBCORES = 16
NUM_WORKERS = NUM_CORES * NUM_SUBCORES
B_PER_W = BATCH // NUM_WORKERS
CHUNK = 128
N_CHUNKS = B_PER_W // CHUNK
LANES = 16
N_SUB_CHUNKS = MAX_SUBWORDS * N_CHUNKS
N_PAD_ROWS = 8


def _body(word_hbm, wt_hbm, sub_hbm, seqj_hbm, out_hbm,
          widx, craw, cidx, acc, tmp_a, tmp_b,
          sem_w, sem_b, sem_s, sem_o):
    wid = lax.axis_index("s") * NUM_CORES + lax.axis_index("c")
    gbase = wid * B_PER_W

    for c in range(N_CHUNKS):
        pltpu.sync_copy(word_hbm.at[pl.ds(gbase + CHUNK * c, CHUNK)],
                        widx.at[c])

    wdescs = [
        pltpu.async_copy(wt_hbm.at[widx.at[c]], acc.at[c], sem_w)
        for c in range(N_CHUNKS)
    ]

    bdescs = []
    for c in range(N_CHUNKS):
        for j in range(MAX_SUBWORDS):
            src = seqj_hbm.at[pl.ds(j * BATCH + gbase + CHUNK * c, CHUNK)]
            bdescs.append(
                pltpu.async_copy(src, craw.at[MAX_SUBWORDS * c + j], sem_b))
    for d in bdescs:
        d.wait()

    iota = lax.iota(jnp.int32, LANES)
    pad_row = jnp.int32(NUM_SUBWORDS) + (iota & jnp.int32(N_PAD_ROWS - 1))
    for rj in range(N_SUB_CHUNKS):
        for t in range(CHUNK // LANES):
            sl = pl.ds(LANES * t, LANES)
            v = craw[rj, sl]
            cidx[rj, sl] = jnp.where(v == 0, pad_row, v)

    bufs = (tmp_a, tmp_b)

    def fire(c, buf):
        return [
            pltpu.async_copy(sub_hbm.at[cidx.at[MAX_SUBWORDS * c + j]],
                             buf.at[j], sem_s)
            for j in range(MAX_SUBWORDS)
        ]

    odescs = []
    pend = fire(0, bufs[0])
    for c in range(N_CHUNKS):
        for d in pend:
            d.wait()
        if c + 1 < N_CHUNKS:
            pend = fire(c + 1, bufs[(c + 1) % 2])
        wdescs[c].wait()
        buf = bufs[c % 2]

        def r_body(r, _, c=c, buf=buf):
            for k in range(D_EMBED // LANES):
                sl = pl.ds(LANES * k, LANES)
                s01 = buf[0, r, sl] + buf[1, r, sl]
                s23 = buf[2, r, sl] + buf[3, r, sl]
                s = s01 + s23 + buf[4, r, sl]
                acc[c, r, sl] = acc[c, r, sl] + s
            return 0

        lax.fori_loop(0, CHUNK, r_body, 0)
        odescs.append(
            pltpu.async_copy(acc.at[c],
                             out_hbm.at[pl.ds(gbase + CHUNK * c, CHUNK)],
                             sem_o))
    for d in odescs:
        d.wait()


_mesh = plsc.VectorSubcoreMesh(core_axis_name="c", subcore_axis_name="s")

_sc_embed = functools.partial(
    pl.kernel,
    out_type=jax.ShapeDtypeStruct((BATCH, D_EMBED), jnp.float32),
    mesh=_mesh,
    scratch_types=[
        pltpu.VMEM((N_CHUNKS, CHUNK), jnp.int32),
        pltpu.VMEM((N_SUB_CHUNKS, CHUNK), jnp.int32),
        pltpu.VMEM((N_SUB_CHUNKS, CHUNK), jnp.int32),
        pltpu.VMEM((N_CHUNKS, CHUNK, D_EMBED), jnp.float32),
        pltpu.VMEM((MAX_SUBWORDS, CHUNK, D_EMBED), jnp.float32),
        pltpu.VMEM((MAX_SUBWORDS, CHUNK, D_EMBED), jnp.float32),
        pltpu.SemaphoreType.DMA,
        pltpu.SemaphoreType.DMA,
        pltpu.SemaphoreType.DMA,
        pltpu.SemaphoreType.DMA,
    ],
    compiler_params=pltpu.CompilerParams(use_tc_tiling_on_sc=False),
)(_body)


@jax.jit
def kernel(word, word_table, subword_table, word_to_subwords):
    word = word.astype(jnp.int32)
    seqj = jnp.take(word_to_subwords, word, axis=0).T.reshape(-1)
    seqj = seqj.astype(jnp.int32)
    sub_ext = jnp.concatenate(
        [subword_table,
         jnp.zeros((N_PAD_ROWS, D_EMBED), subword_table.dtype)], axis=0)
    return _sc_embed(word, word_table, sub_ext, seqj)
